# Initial kernel scaffold; baseline (speedup 1.0000x reference)
import jax
import jax.numpy as jnp
from jax.experimental import pallas as pl


def kernel(dense_inputs, sparse_inputs, seq_inputs, item_inputs, table, W1, b1, a1, W2, b2, a2, W3, b3):
    raise NotImplementedError("write your pallas kernel here")


# speedup vs baseline: 1.2058x; 1.2058x over previous
import functools

import jax
import jax.numpy as jnp
from jax import lax
from jax.experimental import pallas as pl
from jax.experimental.pallas import tpu as pltpu
from jax.experimental.pallas import tpu_sc as plsc


def _sc_gather(table, seq_idx_t, item_idx):
    info = plsc.get_sparse_core_info()
    nw = info.num_cores * info.num_subcores
    n_seq = seq_idx_t.shape[0]
    n_tgt = item_idx.shape[0]
    d = table.shape[1]
    seq_pw = n_seq // nw
    tgt_pw = n_tgt // nw
    ch = 800
    n_ch = seq_pw // ch

    mesh = plsc.VectorSubcoreMesh(core_axis_name="c", subcore_axis_name="s")

    @functools.partial(
        pl.kernel,
        mesh=mesh,
        compiler_params=pltpu.CompilerParams(use_tc_tiling_on_sc=False),
        out_type=(
            jax.ShapeDtypeStruct((n_seq, d), jnp.float32),
            jax.ShapeDtypeStruct((n_tgt, d), jnp.float32),
        ),
        scratch_types=[
            pltpu.VMEM((ch,), jnp.int32),
            pltpu.VMEM((ch, d), jnp.float32),
            pltpu.VMEM((tgt_pw,), jnp.int32),
            pltpu.VMEM((tgt_pw, d), jnp.float32),
            pltpu.SemaphoreType.DMA,
        ],
    )
    def gather_k(table_hbm, seq_idx_hbm, item_idx_hbm, out_seq_hbm,
                 out_tgt_hbm, idx_v, rows_v, tidx_v, trows_v, sem):
        wid = lax.axis_index("s") * info.num_cores + lax.axis_index("c")
        tbase = wid * tgt_pw
        pltpu.sync_copy(item_idx_hbm.at[pl.ds(tbase, tgt_pw)], tidx_v)
        pltpu.async_copy(table_hbm.at[tidx_v], trows_v, sem).wait()
        pltpu.sync_copy(trows_v, out_tgt_hbm.at[pl.ds(tbase, tgt_pw)])
        sbase = wid * seq_pw
        for c in range(n_ch):
            off = sbase + c * ch
            pltpu.sync_copy(seq_idx_hbm.at[pl.ds(off, ch)], idx_v)
            pltpu.async_copy(table_hbm.at[idx_v], rows_v, sem).wait()
            pltpu.sync_copy(rows_v, out_seq_hbm.at[pl.ds(off, ch)])

    return gather_k(table, seq_idx_t, item_idx)


def _tc_din(seq3, idx_t, tgt, wq, wk, wp, b1, a1, w2, b2, a2, w3):
    ll, bb, d = seq3.shape
    h1n = wq.shape[1]
    h2n = w2.shape[1]
    blk = 128
    grid = (bb // blk,)

    def body(seq_ref, idx_ref, tgt_ref, wq_ref, wk_ref, wp_ref, b1_ref,
             a1_ref, w2_ref, b2_ref, a2_ref, w3_ref, out_ref):
        seq = seq_ref[...]
        q = tgt_ref[...]
        k2 = seq.reshape(ll * blk, d)
        qb = jnp.concatenate([q] * ll, axis=0)
        qw = q @ wq_ref[...]
        pre1 = (
            k2 @ wk_ref[...]
            + (qb * k2) @ wp_ref[...]
            + jnp.concatenate([qw] * ll, axis=0)
            + b1_ref[...]
        )
        h1 = jnp.where(pre1 > 0, pre1, a1_ref[...] * pre1)
        pre2 = h1 @ w2_ref[...] + b2_ref[...]
        h2 = jnp.where(pre2 > 0, pre2, a2_ref[...] * pre2)
        s3 = h2.reshape(ll, blk, h2n)
        scores = jnp.sum(s3 * w3_ref[...][None], axis=-1)
        mask = idx_ref[...] != 0
        scores = jnp.where(mask, scores, jnp.float32(-1e9))
        m = jnp.max(scores, axis=0, keepdims=True)
        e = jnp.exp(scores - m)
        attn = e / jnp.sum(e, axis=0, keepdims=True)
        out_ref[...] = jnp.sum(attn[:, :, None] * seq, axis=0)

    full = lambda shape: pl.BlockSpec(shape, lambda i: tuple(0 for _ in shape))
    return pl.pallas_call(
        body,
        grid=grid,
        in_specs=[
            pl.BlockSpec((ll, blk, d), lambda i: (0, i, 0)),
            pl.BlockSpec((ll, blk), lambda i: (0, i)),
            pl.BlockSpec((blk, d), lambda i: (i, 0)),
            full(wq.shape), full(wk.shape), full(wp.shape),
            full(b1.shape), full(a1.shape), full(w2.shape),
            full(b2.shape), full(a2.shape), full(w3.shape),
        ],
        out_specs=pl.BlockSpec((blk, d), lambda i: (i, 0)),
        out_shape=jax.ShapeDtypeStruct((bb, d), jnp.float32),
    )(seq3, idx_t, tgt, wq, wk, wp, b1, a1, w2, b2, a2, w3)


def kernel(dense_inputs, sparse_inputs, seq_inputs, item_inputs, table,
           W1, b1, a1, W2, b2, a2, W3, b3):
    b, l, _ = seq_inputs.shape
    d = table.shape[1]
    idx_t = seq_inputs[:, :, 0].astype(jnp.int32).T
    item_idx = item_inputs[:, 0].astype(jnp.int32)

    seq_rows, tgt_rows = _sc_gather(table, idx_t.reshape(l * b), item_idx)
    seq3 = seq_rows.reshape(l, b, d)

    w1q, w1k, w1d, w1p = W1[:d], W1[d:2 * d], W1[2 * d:3 * d], W1[3 * d:]
    wq = w1q + w1d
    wk = w1k - w1d
    user_info = _tc_din(
        seq3, idx_t, tgt_rows,
        wq, wk, w1p,
        b1.reshape(1, -1), a1.reshape(1, -1),
        W2, b2.reshape(1, -1), a2.reshape(1, -1),
        W3.reshape(1, -1),
    )
    return user_info
